# baseline (speedup 1.0000x reference)
import functools

import jax
import jax.numpy as jnp
from jax import lax
from jax.experimental import pallas as pl
from jax.experimental.pallas import tpu as pltpu
from jax.experimental.pallas import tpu_sc as plsc

_N_NODES = 50000
_E = 3200000
_NC = 2
_NS = 16
_NW = _NC * _NS
_EPW = _E // _NW
_CH = 4000
_NCHUNK = _EPW // _CH
_NB = 2
assert _CH % 80 == 0 and _NCHUNK >= 2 * _NB
_SLICE = 3136
_NPAD = _NS * _SLICE

_mesh = plsc.VectorSubcoreMesh(core_axis_name="c", subcore_axis_name="s")


def _unrolled(total, unroll, body16):
    assert total % unroll == 0
    def outer(kk, carry):
        for u in range(unroll):
            body16(kk * unroll + u)
        return carry
    lax.fori_loop(0, total // unroll, outer, 0)


@functools.partial(
    pl.kernel,
    mesh=_mesh,
    compiler_params=pltpu.CompilerParams(needs_layout_passes=False, skip_device_barrier=True),
    out_type=jax.ShapeDtypeStruct((_NC * _NPAD,), jnp.float32),
    scratch_types=[
        pltpu.VMEM((_NPAD,), jnp.float32),
        [pltpu.VMEM((_CH,), jnp.int32) for _ in range(_NB)],
        [pltpu.VMEM((_CH,), jnp.float32) for _ in range(_NB)],
        [pltpu.VMEM((_SLICE,), jnp.float32) for _ in range(2)],
        pltpu.VMEM((_SLICE,), jnp.float32),
        pltpu.VMEM_SHARED((_NS * _NPAD,), jnp.float32),
        [pltpu.SemaphoreType.DMA for _ in range(_NB)],
        [pltpu.SemaphoreType.DMA for _ in range(2)],
    ],
)
def _hist_kernel(row_hbm, attr_hbm, out_hbm, hist, idxb, attrb, tmps, acc,
                 shared, lsems, rsems):
    c = lax.axis_index("c")
    s = lax.axis_index("s")
    base = (c * _NS + s) * _EPW

    def issue_load(j, p):
        off = base + j * _CH
        pltpu.make_async_copy(row_hbm.at[pl.ds(off, _CH)], idxb[p], lsems[p]).start()
        pltpu.make_async_copy(attr_hbm.at[pl.ds(off, _CH)], attrb[p], lsems[p]).start()

    def wait_load(p):
        pltpu.make_async_copy(row_hbm.at[pl.ds(0, _CH)], idxb[p], lsems[p]).wait()
        pltpu.make_async_copy(attr_hbm.at[pl.ds(0, _CH)], attrb[p], lsems[p]).wait()

    def compute(p):
        @plsc.parallel_loop(0, _CH // 16, unroll=5)
        def vec_body(k):
            i = idxb[p][pl.ds(k * 16, 16)]
            a = attrb[p][pl.ds(k * 16, 16)]
            plsc.addupdate_scatter(hist, [i], a)

    for p in range(_NB):
        issue_load(p, p)

    def zero_body(k):
        hist[pl.ds(k * 16, 16)] = jnp.zeros((16,), jnp.float32)
    _unrolled(_NPAD // 16, 8, zero_body)

    _M = (_NCHUNK - _NB) // _NB

    def round_body(jj, carry):
        for p in range(_NB):
            j = _NB * jj + p
            wait_load(p)
            compute(p)
            issue_load(j + _NB, p)
        return carry
    lax.fori_loop(0, _M, round_body, 0)

    for j in range(_NB * _M, _NCHUNK):
        p = j % _NB
        wait_load(p)
        compute(p)
        if j + _NB < _NCHUNK:
            issue_load(j + _NB, p)

    pltpu.sync_copy(hist, shared.at[pl.ds(s * _NPAD, _NPAD)])
    plsc.subcore_barrier()

    soff = s * _SLICE

    def issue_red(t, q):
        pltpu.make_async_copy(shared.at[pl.ds(t * _NPAD + soff, _SLICE)],
                              tmps[q], rsems[q]).start()

    def wait_red(q):
        pltpu.make_async_copy(shared.at[pl.ds(soff, _SLICE)],
                              tmps[q], rsems[q]).wait()

    issue_red(1, 1)
    pltpu.sync_copy(shared.at[pl.ds(soff, _SLICE)], acc)
    for t in range(1, _NS):
        q = t % 2
        wait_red(q)
        if t + 1 < _NS:
            issue_red(t + 1, (t + 1) % 2)

        def add_body(k, _q=q):
            acc[pl.ds(k * 16, 16)] = acc[pl.ds(k * 16, 16)] + tmps[_q][pl.ds(k * 16, 16)]
        _unrolled(_SLICE // 16, 4, add_body)

    pltpu.sync_copy(acc, out_hbm.at[pl.ds(c * _NPAD + soff, _SLICE)])


@functools.partial(
    pl.kernel,
    mesh=_mesh,
    compiler_params=pltpu.CompilerParams(needs_layout_passes=False, skip_device_barrier=True),
    out_type=jax.ShapeDtypeStruct((_E,), jnp.float32),
    scratch_types=[
        pltpu.VMEM((_NPAD,), jnp.float32),
        [pltpu.VMEM((_CH,), jnp.int32) for _ in range(_NB)],
        [pltpu.VMEM((_CH,), jnp.float32) for _ in range(_NB)],
        [pltpu.VMEM((_CH,), jnp.float32) for _ in range(_NB)],
        [pltpu.VMEM((_SLICE,), jnp.float32) for _ in range(2)],
        pltpu.VMEM_SHARED((_NPAD,), jnp.float32),
        [pltpu.SemaphoreType.DMA for _ in range(_NB)],
        [pltpu.SemaphoreType.DMA for _ in range(_NB)],
    ],
)
def _norm_kernel(row_hbm, attr_hbm, partial_hbm, out_hbm, table,
                 idxb, attrb, outb, slc, table_sh, lsems, ssems):
    c = lax.axis_index("c")
    s = lax.axis_index("s")
    base = (c * _NS + s) * _EPW

    def issue_load(j, p):
        off = base + j * _CH
        pltpu.make_async_copy(row_hbm.at[pl.ds(off, _CH)], idxb[p], lsems[p]).start()
        pltpu.make_async_copy(attr_hbm.at[pl.ds(off, _CH)], attrb[p], lsems[p]).start()

    def wait_load(p):
        pltpu.make_async_copy(row_hbm.at[pl.ds(0, _CH)], idxb[p], lsems[p]).wait()
        pltpu.make_async_copy(attr_hbm.at[pl.ds(0, _CH)], attrb[p], lsems[p]).wait()

    def issue_store(j, p):
        off = base + j * _CH
        pltpu.make_async_copy(outb[p], out_hbm.at[pl.ds(off, _CH)], ssems[p]).start()

    def wait_store(p):
        pltpu.make_async_copy(outb[p], out_hbm.at[pl.ds(0, _CH)], ssems[p]).wait()

    def compute(p):
        @plsc.parallel_loop(0, _CH // 16, unroll=5)
        def vec_body(k):
            i = idxb[p][pl.ds(k * 16, 16)]
            a = attrb[p][pl.ds(k * 16, 16)]
            n = plsc.load_gather(table, [i])
            outb[p][pl.ds(k * 16, 16)] = a * n

    for p in range(_NB):
        issue_load(p, p)

    soff = s * _SLICE
    pltpu.sync_copy(partial_hbm.at[pl.ds(soff, _SLICE)], slc[0])
    pltpu.sync_copy(partial_hbm.at[pl.ds(_NPAD + soff, _SLICE)], slc[1])

    def inv_body(k):
        d = pl.ds(k * 16, 16)
        slc[0][d] = 1.0 / (slc[0][d] + slc[1][d])
    _unrolled(_SLICE // 16, 4, inv_body)

    pltpu.sync_copy(slc[0], table_sh.at[pl.ds(soff, _SLICE)])
    plsc.subcore_barrier()
    pltpu.sync_copy(table_sh, table)

    for p in range(_NB):
        wait_load(p)
        compute(p)
        issue_store(p, p)
        issue_load(p + _NB, p)

    _M = (_NCHUNK - _NB) // _NB

    def round_body(jj, carry):
        for p in range(_NB):
            j = _NB * jj + p
            wait_load(p)
            wait_store(p)
            compute(p)
            issue_store(j, p)
            issue_load(j + _NB, p)
        return carry
    lax.fori_loop(1, _M, round_body, 0)

    for j in range(_NB * _M, _NCHUNK):
        p = j % _NB
        wait_load(p)
        wait_store(p)
        compute(p)
        issue_store(j, p)
        if j + _NB < _NCHUNK:
            issue_load(j + _NB, p)

    for p in range(_NB):
        wait_store(p)


def kernel(edge_index, edge_attr, N):
    row = edge_index[0]
    partial = _hist_kernel(row, edge_attr)
    return _norm_kernel(row, edge_attr, partial)


# speedup vs baseline: 1.0017x; 1.0017x over previous
import functools

import jax
import jax.numpy as jnp
from jax import lax
from jax.experimental import pallas as pl
from jax.experimental.pallas import tpu as pltpu
from jax.experimental.pallas import tpu_sc as plsc

_N_NODES = 50000
_E = 3200000
_NC = 2
_NS = 16
_NW = _NC * _NS
_EPW = _E // _NW
_CH = 4000
_NCHUNK = _EPW // _CH
_NB = 2
assert _CH % 80 == 0 and _NCHUNK >= 2 * _NB
_SLICE = 3136
_NPAD = _NS * _SLICE

_mesh = plsc.VectorSubcoreMesh(core_axis_name="c", subcore_axis_name="s")


def _unrolled(total, unroll, body16):
    assert total % unroll == 0
    def outer(kk, carry):
        for u in range(unroll):
            body16(kk * unroll + u)
        return carry
    lax.fori_loop(0, total // unroll, outer, 0)


@functools.partial(
    pl.kernel,
    mesh=_mesh,
    compiler_params=pltpu.CompilerParams(needs_layout_passes=False),
    out_type=jax.ShapeDtypeStruct((_NC * _NPAD,), jnp.float32),
    scratch_types=[
        pltpu.VMEM((_NPAD,), jnp.float32),
        [pltpu.VMEM((_CH,), jnp.int32) for _ in range(_NB)],
        [pltpu.VMEM((_CH,), jnp.float32) for _ in range(_NB)],
        [pltpu.VMEM((_SLICE,), jnp.float32) for _ in range(2)],
        pltpu.VMEM((_SLICE,), jnp.float32),
        pltpu.VMEM_SHARED((_NS * _NPAD,), jnp.float32),
        [pltpu.SemaphoreType.DMA for _ in range(_NB)],
        [pltpu.SemaphoreType.DMA for _ in range(2)],
    ],
)
def _hist_kernel(row_hbm, attr_hbm, out_hbm, hist, idxb, attrb, tmps, acc,
                 shared, lsems, rsems):
    c = lax.axis_index("c")
    s = lax.axis_index("s")
    base = (c * _NS + s) * _EPW

    def issue_load(j, p):
        off = base + j * _CH
        pltpu.make_async_copy(row_hbm.at[pl.ds(off, _CH)], idxb[p], lsems[p]).start()
        pltpu.make_async_copy(attr_hbm.at[pl.ds(off, _CH)], attrb[p], lsems[p]).start()

    def wait_load(p):
        pltpu.make_async_copy(row_hbm.at[pl.ds(0, _CH)], idxb[p], lsems[p]).wait()
        pltpu.make_async_copy(attr_hbm.at[pl.ds(0, _CH)], attrb[p], lsems[p]).wait()

    def compute(p):
        @plsc.parallel_loop(0, _CH // 16, unroll=5)
        def vec_body(k):
            i = idxb[p][pl.ds(k * 16, 16)]
            a = attrb[p][pl.ds(k * 16, 16)]
            plsc.addupdate_scatter(hist, [i], a)

    for p in range(_NB):
        issue_load(p, p)

    def zero_body(k):
        hist[pl.ds(k * 16, 16)] = jnp.zeros((16,), jnp.float32)
    _unrolled(_NPAD // 16, 8, zero_body)

    _M = (_NCHUNK - _NB) // _NB

    def round_body(jj, carry):
        for p in range(_NB):
            j = _NB * jj + p
            wait_load(p)
            compute(p)
            issue_load(j + _NB, p)
        return carry
    lax.fori_loop(0, _M, round_body, 0)

    for j in range(_NB * _M, _NCHUNK):
        p = j % _NB
        wait_load(p)
        compute(p)
        if j + _NB < _NCHUNK:
            issue_load(j + _NB, p)

    pltpu.sync_copy(hist, shared.at[pl.ds(s * _NPAD, _NPAD)])
    plsc.subcore_barrier()

    soff = s * _SLICE

    def issue_red(t, q):
        pltpu.make_async_copy(shared.at[pl.ds(t * _NPAD + soff, _SLICE)],
                              tmps[q], rsems[q]).start()

    def wait_red(q):
        pltpu.make_async_copy(shared.at[pl.ds(soff, _SLICE)],
                              tmps[q], rsems[q]).wait()

    issue_red(1, 1)
    pltpu.sync_copy(shared.at[pl.ds(soff, _SLICE)], acc)
    for t in range(1, _NS):
        q = t % 2
        wait_red(q)
        if t + 1 < _NS:
            issue_red(t + 1, (t + 1) % 2)

        def add_body(k, _q=q):
            acc[pl.ds(k * 16, 16)] = acc[pl.ds(k * 16, 16)] + tmps[_q][pl.ds(k * 16, 16)]
        _unrolled(_SLICE // 16, 4, add_body)

    pltpu.sync_copy(acc, out_hbm.at[pl.ds(c * _NPAD + soff, _SLICE)])


@functools.partial(
    pl.kernel,
    mesh=_mesh,
    compiler_params=pltpu.CompilerParams(needs_layout_passes=False),
    out_type=jax.ShapeDtypeStruct((_E,), jnp.float32),
    scratch_types=[
        pltpu.VMEM((_NPAD,), jnp.float32),
        [pltpu.VMEM((_CH,), jnp.int32) for _ in range(_NB)],
        [pltpu.VMEM((_CH,), jnp.float32) for _ in range(_NB)],
        [pltpu.VMEM((_CH,), jnp.float32) for _ in range(_NB)],
        [pltpu.VMEM((_SLICE,), jnp.float32) for _ in range(2)],
        pltpu.VMEM_SHARED((_NPAD,), jnp.float32),
        [pltpu.SemaphoreType.DMA for _ in range(_NB)],
        [pltpu.SemaphoreType.DMA for _ in range(_NB)],
    ],
)
def _norm_kernel(row_hbm, attr_hbm, partial_hbm, out_hbm, table,
                 idxb, attrb, outb, slc, table_sh, lsems, ssems):
    c = lax.axis_index("c")
    s = lax.axis_index("s")
    base = (c * _NS + s) * _EPW

    def issue_load(j, p):
        off = base + j * _CH
        pltpu.make_async_copy(row_hbm.at[pl.ds(off, _CH)], idxb[p], lsems[p]).start()
        pltpu.make_async_copy(attr_hbm.at[pl.ds(off, _CH)], attrb[p], lsems[p]).start()

    def wait_load(p):
        pltpu.make_async_copy(row_hbm.at[pl.ds(0, _CH)], idxb[p], lsems[p]).wait()
        pltpu.make_async_copy(attr_hbm.at[pl.ds(0, _CH)], attrb[p], lsems[p]).wait()

    def issue_store(j, p):
        off = base + j * _CH
        pltpu.make_async_copy(outb[p], out_hbm.at[pl.ds(off, _CH)], ssems[p]).start()

    def wait_store(p):
        pltpu.make_async_copy(outb[p], out_hbm.at[pl.ds(0, _CH)], ssems[p]).wait()

    def compute(p):
        @plsc.parallel_loop(0, _CH // 16, unroll=5)
        def vec_body(k):
            i = idxb[p][pl.ds(k * 16, 16)]
            a = attrb[p][pl.ds(k * 16, 16)]
            n = plsc.load_gather(table, [i])
            outb[p][pl.ds(k * 16, 16)] = a * n

    for p in range(_NB):
        issue_load(p, p)

    soff = s * _SLICE
    pltpu.sync_copy(partial_hbm.at[pl.ds(soff, _SLICE)], slc[0])
    pltpu.sync_copy(partial_hbm.at[pl.ds(_NPAD + soff, _SLICE)], slc[1])

    def inv_body(k):
        d = pl.ds(k * 16, 16)
        slc[0][d] = 1.0 / (slc[0][d] + slc[1][d])
    _unrolled(_SLICE // 16, 4, inv_body)

    pltpu.sync_copy(slc[0], table_sh.at[pl.ds(soff, _SLICE)])
    plsc.subcore_barrier()
    pltpu.sync_copy(table_sh, table)

    for p in range(_NB):
        wait_load(p)
        compute(p)
        issue_store(p, p)
        issue_load(p + _NB, p)

    _M = (_NCHUNK - _NB) // _NB

    def round_body(jj, carry):
        for p in range(_NB):
            j = _NB * jj + p
            wait_load(p)
            wait_store(p)
            compute(p)
            issue_store(j, p)
            issue_load(j + _NB, p)
        return carry
    lax.fori_loop(1, _M, round_body, 0)

    for j in range(_NB * _M, _NCHUNK):
        p = j % _NB
        wait_load(p)
        wait_store(p)
        compute(p)
        issue_store(j, p)
        if j + _NB < _NCHUNK:
            issue_load(j + _NB, p)

    for p in range(_NB):
        wait_store(p)


def kernel(edge_index, edge_attr, N):
    row = edge_index[0]
    partial = _hist_kernel(row, edge_attr)
    return _norm_kernel(row, edge_attr, partial)
